# baseline (speedup 1.0000x reference)
import functools

import jax
import jax.numpy as jnp
from jax import lax
from jax.experimental import pallas as pl
from jax.experimental.pallas import tpu as pltpu

try:
    from jax.experimental.pallas import tpu_sc as plsc
except ImportError:
    plsc = None

_L = 16


def _sc_dirs(vert_flat, idx_flat, v, n):
    info = plsc.get_sparse_core_info()
    nw = info.num_cores * info.num_subcores
    num_rows = idx_flat.shape[0]
    per_w = num_rows // nw
    steps = per_w // _L
    verts_per_w = per_w // n
    entries_per_batch = v * n
    workers_per_batch = entries_per_batch // per_w
    mesh = plsc.VectorSubcoreMesh(core_axis_name="c", subcore_axis_name="s")
    plane = jax.ShapeDtypeStruct((num_rows,), jnp.float32)

    @functools.partial(
        pl.kernel,
        mesh=mesh,
        out_type=(plane, plane, plane),
        compiler_params=pltpu.CompilerParams(needs_layout_passes=False),
        scratch_types=[
            pltpu.VMEM((vert_flat.shape[0],), jnp.float32),
            pltpu.VMEM((per_w,), jnp.int32),
            pltpu.VMEM((per_w,), jnp.float32),
            pltpu.VMEM((per_w,), jnp.float32),
            pltpu.VMEM((per_w,), jnp.float32),
        ],
    )
    def k(vert_hbm, idx_hbm, ox_hbm, oy_hbm, oz_hbm,
          vert_v, idx_v, dx_v, dy_v, dz_v):
        wid = lax.axis_index("s") * info.num_cores + lax.axis_index("c")
        base = wid * per_w
        badd = (wid // workers_per_batch) * v
        pltpu.sync_copy(vert_hbm, vert_v)
        pltpu.sync_copy(idx_hbm.at[pl.ds(base, per_w)], idx_v)

        def body(i, _):
            off = i * _L
            iv3 = (idx_v[pl.ds(off, _L)] + badd) * 3
            gx = plsc.load_gather(vert_v, [iv3])
            gy = plsc.load_gather(vert_v, [iv3 + 1])
            gz = plsc.load_gather(vert_v, [iv3 + 2])
            c3 = jnp.full((_L,), 3 * (wid * verts_per_w), jnp.int32) + i * 3
            dx_v[pl.ds(off, _L)] = gx - plsc.load_gather(vert_v, [c3])
            dy_v[pl.ds(off, _L)] = gy - plsc.load_gather(vert_v, [c3 + 1])
            dz_v[pl.ds(off, _L)] = gz - plsc.load_gather(vert_v, [c3 + 2])
            return 0

        lax.fori_loop(0, steps, body, 0)
        pltpu.sync_copy(dx_v, ox_hbm.at[pl.ds(base, per_w)])
        pltpu.sync_copy(dy_v, oy_hbm.at[pl.ds(base, per_w)])
        pltpu.sync_copy(dz_v, oz_hbm.at[pl.ds(base, per_w)])

    return k(vert_flat, idx_flat)


def _rsqrt_refined(s):
    r = lax.rsqrt(jnp.maximum(s, 1e-30))
    r = r * (1.5 - 0.5 * s * r * r)
    return jnp.where(s > 0.0, r, 0.0)


def _tc_body(x_ref, y_ref, z_ref, fm_ref, supt_ref, bias_ref, out_ref,
             sup_scr):
    @pl.when(pl.program_id(0) == 0)
    def _():
        supt = supt_ref[...]
        sxc = supt[:, 0:1]
        syc = supt[:, 1:2]
        szc = supt[:, 2:3]
        s2 = sxc * sxc + syc * syc + szc * szc
        sinv = _rsqrt_refined(s2)
        sup_scr[:, 0:1] = sxc * sinv
        sup_scr[:, 1:2] = syc * sinv
        sup_scr[:, 2:3] = szc * sinv

    x = x_ref[...].T
    y = y_ref[...].T
    z = z_ref[...].T
    s = x * x + y * y + z * z
    inv = _rsqrt_refined(s)
    xn = x * inv
    yn = y * inv
    zn = z * inv

    fmt = fm_ref[...].T
    n = x.shape[0]
    o = fmt.shape[0]
    s_num = supt_ref.shape[0] // o
    cols = []
    for t in range(s_num):
        sx_t = sup_scr[t * o:(t + 1) * o, 0:1]
        sy_t = sup_scr[t * o:(t + 1) * o, 1:2]
        sz_t = sup_scr[t * o:(t + 1) * o, 2:3]
        m = None
        for j in range(n):
            th = xn[j:j + 1] * sx_t + yn[j:j + 1] * sy_t + zn[j:j + 1] * sz_t
            m = th if m is None else jnp.maximum(m, th)
        m = jnp.maximum(m, 0.0)
        cols.append(jnp.sum(fmt * m, axis=0, keepdims=True))
    out_t = jnp.concatenate(cols, axis=0)
    out_ref[...] = out_t.T + bias_ref[...]


def _tc_compute(xt, yt, zt, fm, supt_pad, bias2d, rows_block=128):
    bv, n = xt.shape
    o = fm.shape[1]
    s_num = bias2d.shape[1]
    dir_spec = pl.BlockSpec((rows_block, n), lambda i: (i, 0))
    return pl.pallas_call(
        _tc_body,
        grid=(bv // rows_block,),
        in_specs=[
            dir_spec,
            dir_spec,
            dir_spec,
            pl.BlockSpec((rows_block, o), lambda i: (i, 0)),
            pl.BlockSpec((s_num * o, 8), lambda i: (0, 0)),
            pl.BlockSpec((1, s_num), lambda i: (0, 0)),
        ],
        out_specs=pl.BlockSpec((rows_block, s_num), lambda i: (i, 0)),
        out_shape=jax.ShapeDtypeStruct((bv, s_num), jnp.float32),
        scratch_shapes=[pltpu.VMEM((s_num * o, 8), jnp.float32)],
    )(xt, yt, zt, fm, supt_pad, bias2d)


def kernel(neighbor_index, vertices, feature_map, directions, bias):
    bs, v, n = neighbor_index.shape
    o = feature_map.shape[-1]
    s_num = directions.shape[1] // o
    bv = bs * v

    dx, dy, dz = _sc_dirs(vertices.reshape(-1),
                          neighbor_index.astype(jnp.int32).reshape(-1), v, n)

    fm = feature_map.reshape(bv, o)
    supt_pad = jnp.zeros((s_num * o, 8), jnp.float32).at[:, :3].set(
        directions.T)
    bias2d = bias.reshape(1, s_num)

    out = _tc_compute(dx.reshape(bv, n), dy.reshape(bv, n),
                      dz.reshape(bv, n), fm, supt_pad, bias2d)
    return out.reshape(bs, v, s_num)


# speedup vs baseline: 1.5092x; 1.5092x over previous
import functools

import jax
import jax.numpy as jnp
from jax import lax
from jax.experimental import pallas as pl
from jax.experimental.pallas import tpu as pltpu

try:
    from jax.experimental.pallas import tpu_sc as plsc
except ImportError:
    plsc = None

_L = 16


def _sc_dirs(xs, ys, zs, gidx_j, num_rows):
    info = plsc.get_sparse_core_info()
    nw = info.num_cores * info.num_subcores
    per_w = num_rows // nw
    steps = per_w // _L
    bv = xs.shape[0]
    mesh = plsc.VectorSubcoreMesh(core_axis_name="c", subcore_axis_name="s")
    plane = jax.ShapeDtypeStruct((num_rows,), jnp.float32)

    @functools.partial(
        pl.kernel,
        mesh=mesh,
        out_type=(plane, plane, plane),
        compiler_params=pltpu.CompilerParams(needs_layout_passes=False),
        scratch_types=[
            pltpu.VMEM((bv,), jnp.float32),
            pltpu.VMEM((bv,), jnp.float32),
            pltpu.VMEM((bv,), jnp.float32),
            pltpu.VMEM((per_w,), jnp.int32),
            pltpu.VMEM((per_w,), jnp.float32),
            pltpu.VMEM((per_w,), jnp.float32),
            pltpu.VMEM((per_w,), jnp.float32),
        ],
    )
    def k(xs_hbm, ys_hbm, zs_hbm, idx_hbm, ox_hbm, oy_hbm, oz_hbm,
          xs_v, ys_v, zs_v, idx_v, dx_v, dy_v, dz_v):
        wid = lax.axis_index("s") * info.num_cores + lax.axis_index("c")
        base = wid * per_w
        gbase = lax.rem(base, bv)
        pltpu.sync_copy(xs_hbm, xs_v)
        pltpu.sync_copy(ys_hbm, ys_v)
        pltpu.sync_copy(zs_hbm, zs_v)
        pltpu.sync_copy(idx_hbm.at[pl.ds(base, per_w)], idx_v)

        def body(i, _):
            off = i * _L
            iv = idx_v[pl.ds(off, _L)]
            gx = plsc.load_gather(xs_v, [iv])
            gy = plsc.load_gather(ys_v, [iv])
            gz = plsc.load_gather(zs_v, [iv])
            coff = gbase + off
            dx_v[pl.ds(off, _L)] = gx - xs_v[pl.ds(coff, _L)]
            dy_v[pl.ds(off, _L)] = gy - ys_v[pl.ds(coff, _L)]
            dz_v[pl.ds(off, _L)] = gz - zs_v[pl.ds(coff, _L)]
            return 0

        lax.fori_loop(0, steps, body, 0)
        pltpu.sync_copy(dx_v, ox_hbm.at[pl.ds(base, per_w)])
        pltpu.sync_copy(dy_v, oy_hbm.at[pl.ds(base, per_w)])
        pltpu.sync_copy(dz_v, oz_hbm.at[pl.ds(base, per_w)])

    return k(xs, ys, zs, gidx_j)


def _rsqrt_refined(s):
    r = lax.rsqrt(jnp.maximum(s, 1e-30))
    r = r * (1.5 - 0.5 * s * r * r)
    return jnp.where(s > 0.0, r, 0.0)


def _tc_body(x_ref, y_ref, z_ref, fm_ref, supt_ref, bias_ref, out_ref,
             sup_scr):
    @pl.when(pl.program_id(0) == 0)
    def _():
        supt = supt_ref[...]
        sxc = supt[:, 0:1]
        syc = supt[:, 1:2]
        szc = supt[:, 2:3]
        s2 = sxc * sxc + syc * syc + szc * szc
        sinv = _rsqrt_refined(s2)
        sup_scr[:, 0:1] = sxc * sinv
        sup_scr[:, 1:2] = syc * sinv
        sup_scr[:, 2:3] = szc * sinv

    x = x_ref[...]
    y = y_ref[...]
    z = z_ref[...]
    s = x * x + y * y + z * z
    inv = _rsqrt_refined(s)
    xn = x * inv
    yn = y * inv
    zn = z * inv

    xnb = xn.astype(jnp.bfloat16)
    ynb = yn.astype(jnp.bfloat16)
    znb = zn.astype(jnp.bfloat16)

    fmt = fm_ref[...].T
    n = x.shape[0]
    o = fmt.shape[0]
    s_num = supt_ref.shape[0] // o
    cols = []
    for t in range(s_num):
        sx_t = sup_scr[t * o:(t + 1) * o, 0:1].astype(jnp.bfloat16)
        sy_t = sup_scr[t * o:(t + 1) * o, 1:2].astype(jnp.bfloat16)
        sz_t = sup_scr[t * o:(t + 1) * o, 2:3].astype(jnp.bfloat16)
        m = None
        for j in range(n):
            th = (xnb[j:j + 1] * sx_t + ynb[j:j + 1] * sy_t
                  + znb[j:j + 1] * sz_t)
            m = th if m is None else jnp.maximum(m, th)
        m = jnp.maximum(m.astype(jnp.float32), 0.0)
        cols.append(jnp.sum(fmt * m, axis=0, keepdims=True))
    out_t = jnp.concatenate(cols, axis=0)
    out_ref[...] = out_t.T + bias_ref[...]


def _tc_compute(xt, yt, zt, fm, supt_pad, bias2d, rows_block=128):
    n, bv = xt.shape
    o = fm.shape[1]
    s_num = bias2d.shape[1]
    dir_spec = pl.BlockSpec((n, rows_block), lambda i: (0, i))
    return pl.pallas_call(
        _tc_body,
        grid=(bv // rows_block,),
        in_specs=[
            dir_spec,
            dir_spec,
            dir_spec,
            pl.BlockSpec((rows_block, o), lambda i: (i, 0)),
            pl.BlockSpec((s_num * o, 8), lambda i: (0, 0)),
            pl.BlockSpec((1, s_num), lambda i: (0, 0)),
        ],
        out_specs=pl.BlockSpec((rows_block, s_num), lambda i: (i, 0)),
        out_shape=jax.ShapeDtypeStruct((bv, s_num), jnp.float32),
        scratch_shapes=[pltpu.VMEM((s_num * o, 8), jnp.float32)],
    )(xt, yt, zt, fm, supt_pad, bias2d)


def kernel(neighbor_index, vertices, feature_map, directions, bias):
    bs, v, n = neighbor_index.shape
    o = feature_map.shape[-1]
    s_num = directions.shape[1] // o
    bv = bs * v

    idx = neighbor_index.astype(jnp.int32)
    gidx = idx + (jnp.arange(bs, dtype=jnp.int32) * v)[:, None, None]
    gidx_j = jnp.transpose(gidx, (2, 0, 1)).reshape(-1)

    vflat = vertices.reshape(bv, 3)
    dx, dy, dz = _sc_dirs(vflat[:, 0], vflat[:, 1], vflat[:, 2],
                          gidx_j, n * bv)

    fm = feature_map.reshape(bv, o)
    supt_pad = jnp.zeros((s_num * o, 8), jnp.float32).at[:, :3].set(
        directions.T)
    bias2d = bias.reshape(1, s_num)

    out = _tc_compute(dx.reshape(n, bv), dy.reshape(n, bv),
                      dz.reshape(n, bv), fm, supt_pad, bias2d)
    return out.reshape(bs, v, s_num)


# speedup vs baseline: 1.5871x; 1.0516x over previous
import functools

import jax
import jax.numpy as jnp
from jax import lax
from jax.experimental import pallas as pl
from jax.experimental.pallas import tpu as pltpu

try:
    from jax.experimental.pallas import tpu_sc as plsc
except ImportError:
    plsc = None

_L = 16


def _sc_dirs(xs, ys, zs, gidx_j, num_rows):
    info = plsc.get_sparse_core_info()
    nw = info.num_cores * info.num_subcores
    per_w = num_rows // nw
    steps = per_w // _L
    bv = xs.shape[0]
    mesh = plsc.VectorSubcoreMesh(core_axis_name="c", subcore_axis_name="s")
    plane = jax.ShapeDtypeStruct((num_rows,), jnp.float32)

    @functools.partial(
        pl.kernel,
        mesh=mesh,
        out_type=(plane, plane, plane),
        compiler_params=pltpu.CompilerParams(needs_layout_passes=False),
        scratch_types=[
            pltpu.VMEM((bv,), jnp.float32),
            pltpu.VMEM((bv,), jnp.float32),
            pltpu.VMEM((bv,), jnp.float32),
            pltpu.VMEM((per_w,), jnp.int32),
            pltpu.VMEM((per_w,), jnp.float32),
            pltpu.VMEM((per_w,), jnp.float32),
            pltpu.VMEM((per_w,), jnp.float32),
        ],
    )
    def k(xs_hbm, ys_hbm, zs_hbm, idx_hbm, ox_hbm, oy_hbm, oz_hbm,
          xs_v, ys_v, zs_v, idx_v, dx_v, dy_v, dz_v):
        wid = lax.axis_index("s") * info.num_cores + lax.axis_index("c")
        base = wid * per_w
        gbase = lax.rem(base, bv)
        pltpu.sync_copy(xs_hbm, xs_v)
        pltpu.sync_copy(ys_hbm, ys_v)
        pltpu.sync_copy(zs_hbm, zs_v)
        pltpu.sync_copy(idx_hbm.at[pl.ds(base, per_w)], idx_v)

        def body(i, _):
            off = i * _L
            iv = idx_v[pl.ds(off, _L)]
            gx = plsc.load_gather(xs_v, [iv])
            gy = plsc.load_gather(ys_v, [iv])
            gz = plsc.load_gather(zs_v, [iv])
            coff = gbase + off
            dx_v[pl.ds(off, _L)] = gx - xs_v[pl.ds(coff, _L)]
            dy_v[pl.ds(off, _L)] = gy - ys_v[pl.ds(coff, _L)]
            dz_v[pl.ds(off, _L)] = gz - zs_v[pl.ds(coff, _L)]
            return 0

        lax.fori_loop(0, steps, body, 0)
        pltpu.sync_copy(dx_v, ox_hbm.at[pl.ds(base, per_w)])
        pltpu.sync_copy(dy_v, oy_hbm.at[pl.ds(base, per_w)])
        pltpu.sync_copy(dz_v, oz_hbm.at[pl.ds(base, per_w)])

    return k(xs, ys, zs, gidx_j)


def _rsqrt_refined(s):
    r = lax.rsqrt(jnp.maximum(s, 1e-30))
    r = r * (1.5 - 0.5 * s * r * r)
    return jnp.where(s > 0.0, r, 0.0)


def _tc_body(x_ref, y_ref, z_ref, fm_ref, supt_ref, bias_ref, out_ref,
             sup_scr):
    @pl.when(pl.program_id(0) == 0)
    def _():
        supt = supt_ref[...]
        sxc = supt[:, 0:1]
        syc = supt[:, 1:2]
        szc = supt[:, 2:3]
        s2 = sxc * sxc + syc * syc + szc * szc
        sinv = _rsqrt_refined(s2)
        sup_scr[:, 0:1] = sxc * sinv
        sup_scr[:, 1:2] = syc * sinv
        sup_scr[:, 2:3] = szc * sinv

    x = x_ref[...]
    y = y_ref[...]
    z = z_ref[...]
    s = x * x + y * y + z * z
    inv = _rsqrt_refined(s)
    xn = x * inv
    yn = y * inv
    zn = z * inv

    xnb = xn.astype(jnp.bfloat16)
    ynb = yn.astype(jnp.bfloat16)
    znb = zn.astype(jnp.bfloat16)

    fmt = fm_ref[...].T
    n = x.shape[0]
    o = fmt.shape[0]
    s_num = supt_ref.shape[0] // o
    cols = []
    for t in range(s_num):
        sx_t = sup_scr[t * o:(t + 1) * o, 0:1].astype(jnp.bfloat16)
        sy_t = sup_scr[t * o:(t + 1) * o, 1:2].astype(jnp.bfloat16)
        sz_t = sup_scr[t * o:(t + 1) * o, 2:3].astype(jnp.bfloat16)
        m = None
        for j in range(n):
            th = (xnb[j:j + 1] * sx_t + ynb[j:j + 1] * sy_t
                  + znb[j:j + 1] * sz_t)
            m = th if m is None else jnp.maximum(m, th)
        m = jnp.maximum(m.astype(jnp.float32), 0.0)
        cols.append(jnp.sum(fmt * m, axis=0, keepdims=True))
    out_t = jnp.concatenate(cols, axis=0)
    out_ref[...] = out_t.T + bias_ref[...]


def _tc_compute(xt, yt, zt, fm, supt_pad, bias2d, rows_block=256):
    n, bv = xt.shape
    o = fm.shape[1]
    s_num = bias2d.shape[1]
    dir_spec = pl.BlockSpec((n, rows_block), lambda i: (0, i))
    return pl.pallas_call(
        _tc_body,
        grid=(bv // rows_block,),
        in_specs=[
            dir_spec,
            dir_spec,
            dir_spec,
            pl.BlockSpec((rows_block, o), lambda i: (i, 0)),
            pl.BlockSpec((s_num * o, 8), lambda i: (0, 0)),
            pl.BlockSpec((1, s_num), lambda i: (0, 0)),
        ],
        out_specs=pl.BlockSpec((rows_block, s_num), lambda i: (i, 0)),
        out_shape=jax.ShapeDtypeStruct((bv, s_num), jnp.float32),
        scratch_shapes=[pltpu.VMEM((s_num * o, 8), jnp.float32)],
    )(xt, yt, zt, fm, supt_pad, bias2d)


def kernel(neighbor_index, vertices, feature_map, directions, bias):
    bs, v, n = neighbor_index.shape
    o = feature_map.shape[-1]
    s_num = directions.shape[1] // o
    bv = bs * v

    idx = neighbor_index.astype(jnp.int32)
    gidx = idx + (jnp.arange(bs, dtype=jnp.int32) * v)[:, None, None]
    gidx_j = jnp.transpose(gidx, (2, 0, 1)).reshape(-1)

    vflat = vertices.reshape(bv, 3)
    dx, dy, dz = _sc_dirs(vflat[:, 0], vflat[:, 1], vflat[:, 2],
                          gidx_j, n * bv)

    fm = feature_map.reshape(bv, o)
    supt_pad = jnp.zeros((s_num * o, 8), jnp.float32).at[:, :3].set(
        directions.T)
    bias2d = bias.reshape(1, s_num)

    out = _tc_compute(dx.reshape(n, bv), dy.reshape(n, bv),
                      dz.reshape(n, bv), fm, supt_pad, bias2d)
    return out.reshape(bs, v, s_num)


# speedup vs baseline: 1.6288x; 1.0263x over previous
import functools

import jax
import jax.numpy as jnp
from jax import lax
from jax.experimental import pallas as pl
from jax.experimental.pallas import tpu as pltpu

try:
    from jax.experimental.pallas import tpu_sc as plsc
except ImportError:
    plsc = None

_L = 16


def _sc_dirs(xs, ys, zs, gidx_j, num_rows):
    info = plsc.get_sparse_core_info()
    nw = info.num_cores * info.num_subcores
    per_w = num_rows // nw
    steps = per_w // _L
    bv = xs.shape[0]
    mesh = plsc.VectorSubcoreMesh(core_axis_name="c", subcore_axis_name="s")
    plane = jax.ShapeDtypeStruct((num_rows,), jnp.float32)

    @functools.partial(
        pl.kernel,
        mesh=mesh,
        out_type=(plane, plane, plane),
        compiler_params=pltpu.CompilerParams(needs_layout_passes=False),
        scratch_types=[
            pltpu.VMEM((bv,), jnp.float32),
            pltpu.VMEM((bv,), jnp.float32),
            pltpu.VMEM((bv,), jnp.float32),
            pltpu.VMEM((per_w,), jnp.int32),
            pltpu.VMEM((per_w,), jnp.float32),
            pltpu.VMEM((per_w,), jnp.float32),
            pltpu.VMEM((per_w,), jnp.float32),
        ],
    )
    def k(xs_hbm, ys_hbm, zs_hbm, idx_hbm, ox_hbm, oy_hbm, oz_hbm,
          xs_v, ys_v, zs_v, idx_v, dx_v, dy_v, dz_v):
        wid = lax.axis_index("s") * info.num_cores + lax.axis_index("c")
        base = wid * per_w
        gbase = lax.rem(base, bv)
        pltpu.sync_copy(xs_hbm, xs_v)
        pltpu.sync_copy(ys_hbm, ys_v)
        pltpu.sync_copy(zs_hbm, zs_v)
        pltpu.sync_copy(idx_hbm.at[pl.ds(base, per_w)], idx_v)

        def body(i, _):
            off = i * _L
            iv = idx_v[pl.ds(off, _L)]
            gx = plsc.load_gather(xs_v, [iv])
            gy = plsc.load_gather(ys_v, [iv])
            gz = plsc.load_gather(zs_v, [iv])
            coff = gbase + off
            dx_v[pl.ds(off, _L)] = gx - xs_v[pl.ds(coff, _L)]
            dy_v[pl.ds(off, _L)] = gy - ys_v[pl.ds(coff, _L)]
            dz_v[pl.ds(off, _L)] = gz - zs_v[pl.ds(coff, _L)]
            return 0

        lax.fori_loop(0, steps, body, 0)
        pltpu.sync_copy(dx_v, ox_hbm.at[pl.ds(base, per_w)])
        pltpu.sync_copy(dy_v, oy_hbm.at[pl.ds(base, per_w)])
        pltpu.sync_copy(dz_v, oz_hbm.at[pl.ds(base, per_w)])

    return k(xs, ys, zs, gidx_j)


def _rsqrt_refined(s):
    r = lax.rsqrt(jnp.maximum(s, 1e-30))
    r = r * (1.5 - 0.5 * s * r * r)
    return jnp.where(s > 0.0, r, 0.0)


def _tc_body(x_ref, y_ref, z_ref, fm_ref, supt_ref, bias_ref, out_ref,
             sup_scr):
    @pl.when(pl.program_id(0) == 0)
    def _():
        supt = supt_ref[...]
        sxc = supt[:, 0:1]
        syc = supt[:, 1:2]
        szc = supt[:, 2:3]
        s2 = sxc * sxc + syc * syc + szc * szc
        sinv = _rsqrt_refined(s2)
        sup_scr[:, 0:1] = sxc * sinv
        sup_scr[:, 1:2] = syc * sinv
        sup_scr[:, 2:3] = szc * sinv

    x = x_ref[...]
    y = y_ref[...]
    z = z_ref[...]
    s = x * x + y * y + z * z
    inv = _rsqrt_refined(s)
    xn = x * inv
    yn = y * inv
    zn = z * inv

    xnb = xn.astype(jnp.bfloat16)
    ynb = yn.astype(jnp.bfloat16)
    znb = zn.astype(jnp.bfloat16)

    fmt = fm_ref[...].T
    n = x.shape[0]
    o = fmt.shape[0]
    s_num = supt_ref.shape[0] // o
    cols = []
    for t in range(s_num):
        sx_t = sup_scr[t * o:(t + 1) * o, 0:1].astype(jnp.bfloat16)
        sy_t = sup_scr[t * o:(t + 1) * o, 1:2].astype(jnp.bfloat16)
        sz_t = sup_scr[t * o:(t + 1) * o, 2:3].astype(jnp.bfloat16)
        m = None
        for j in range(n):
            th = (xnb[j:j + 1] * sx_t + ynb[j:j + 1] * sy_t
                  + znb[j:j + 1] * sz_t)
            m = th if m is None else jnp.maximum(m, th)
        m = jnp.maximum(m.astype(jnp.float32), 0.0)
        cols.append(jnp.sum(fmt * m, axis=0, keepdims=True))
    out_t = jnp.concatenate(cols, axis=0)
    out_ref[...] = out_t.T + bias_ref[...]


def _tc_compute(xt, yt, zt, fm, supt_pad, bias2d, rows_block=512):
    n, bv = xt.shape
    o = fm.shape[1]
    s_num = bias2d.shape[1]
    dir_spec = pl.BlockSpec((n, rows_block), lambda i: (0, i))
    return pl.pallas_call(
        _tc_body,
        grid=(bv // rows_block,),
        in_specs=[
            dir_spec,
            dir_spec,
            dir_spec,
            pl.BlockSpec((rows_block, o), lambda i: (i, 0)),
            pl.BlockSpec((s_num * o, 8), lambda i: (0, 0)),
            pl.BlockSpec((1, s_num), lambda i: (0, 0)),
        ],
        out_specs=pl.BlockSpec((rows_block, s_num), lambda i: (i, 0)),
        out_shape=jax.ShapeDtypeStruct((bv, s_num), jnp.float32),
        scratch_shapes=[pltpu.VMEM((s_num * o, 8), jnp.float32)],
    )(xt, yt, zt, fm, supt_pad, bias2d)


def kernel(neighbor_index, vertices, feature_map, directions, bias):
    bs, v, n = neighbor_index.shape
    o = feature_map.shape[-1]
    s_num = directions.shape[1] // o
    bv = bs * v

    idx = neighbor_index.astype(jnp.int32)
    gidx = idx + (jnp.arange(bs, dtype=jnp.int32) * v)[:, None, None]
    gidx_j = jnp.transpose(gidx, (2, 0, 1)).reshape(-1)

    vflat = vertices.reshape(bv, 3)
    dx, dy, dz = _sc_dirs(vflat[:, 0], vflat[:, 1], vflat[:, 2],
                          gidx_j, n * bv)

    fm = feature_map.reshape(bv, o)
    supt_pad = jnp.zeros((s_num * o, 8), jnp.float32).at[:, :3].set(
        directions.T)
    bias2d = bias.reshape(1, s_num)

    out = _tc_compute(dx.reshape(n, bv), dy.reshape(n, bv),
                      dz.reshape(n, bv), fm, supt_pad, bias2d)
    return out.reshape(bs, v, s_num)


# speedup vs baseline: 1.6602x; 1.0193x over previous
import functools

import jax
import jax.numpy as jnp
from jax import lax
from jax.experimental import pallas as pl
from jax.experimental.pallas import tpu as pltpu

try:
    from jax.experimental.pallas import tpu_sc as plsc
except ImportError:
    plsc = None

_L = 16


def _sc_dirs(xs, ys, zs, gidx_j, num_rows):
    info = plsc.get_sparse_core_info()
    nw = info.num_cores * info.num_subcores
    per_w = num_rows // nw
    steps = per_w // _L
    bv = xs.shape[0]
    mesh = plsc.VectorSubcoreMesh(core_axis_name="c", subcore_axis_name="s")
    plane = jax.ShapeDtypeStruct((num_rows,), jnp.float32)

    @functools.partial(
        pl.kernel,
        mesh=mesh,
        out_type=(plane, plane, plane),
        compiler_params=pltpu.CompilerParams(needs_layout_passes=False),
        scratch_types=[
            pltpu.VMEM((bv,), jnp.float32),
            pltpu.VMEM((bv,), jnp.float32),
            pltpu.VMEM((bv,), jnp.float32),
            pltpu.VMEM((per_w,), jnp.int32),
            pltpu.VMEM((per_w,), jnp.float32),
            pltpu.VMEM((per_w,), jnp.float32),
            pltpu.VMEM((per_w,), jnp.float32),
        ],
    )
    def k(xs_hbm, ys_hbm, zs_hbm, idx_hbm, ox_hbm, oy_hbm, oz_hbm,
          xs_v, ys_v, zs_v, idx_v, dx_v, dy_v, dz_v):
        wid = lax.axis_index("s") * info.num_cores + lax.axis_index("c")
        base = wid * per_w
        gbase = lax.rem(base, bv)
        pltpu.sync_copy(xs_hbm, xs_v)
        pltpu.sync_copy(ys_hbm, ys_v)
        pltpu.sync_copy(zs_hbm, zs_v)
        pltpu.sync_copy(idx_hbm.at[pl.ds(base, per_w)], idx_v)

        @functools.partial(plsc.parallel_loop(0, steps, unroll=4))
        def _(i):
            off = i * _L
            iv = idx_v[pl.ds(off, _L)]
            gx = plsc.load_gather(xs_v, [iv])
            gy = plsc.load_gather(ys_v, [iv])
            gz = plsc.load_gather(zs_v, [iv])
            coff = gbase + off
            dx_v[pl.ds(off, _L)] = gx - xs_v[pl.ds(coff, _L)]
            dy_v[pl.ds(off, _L)] = gy - ys_v[pl.ds(coff, _L)]
            dz_v[pl.ds(off, _L)] = gz - zs_v[pl.ds(coff, _L)]
        pltpu.sync_copy(dx_v, ox_hbm.at[pl.ds(base, per_w)])
        pltpu.sync_copy(dy_v, oy_hbm.at[pl.ds(base, per_w)])
        pltpu.sync_copy(dz_v, oz_hbm.at[pl.ds(base, per_w)])

    return k(xs, ys, zs, gidx_j)


def _rsqrt_refined(s):
    r = lax.rsqrt(jnp.maximum(s, 1e-30))
    r = r * (1.5 - 0.5 * s * r * r)
    return jnp.where(s > 0.0, r, 0.0)


def _tc_body(x_ref, y_ref, z_ref, fm_ref, supt_ref, bias_ref, out_ref,
             sup_scr):
    @pl.when(pl.program_id(0) == 0)
    def _():
        supt = supt_ref[...]
        sxc = supt[:, 0:1]
        syc = supt[:, 1:2]
        szc = supt[:, 2:3]
        s2 = sxc * sxc + syc * syc + szc * szc
        sinv = _rsqrt_refined(s2)
        sup_scr[:, 0:1] = sxc * sinv
        sup_scr[:, 1:2] = syc * sinv
        sup_scr[:, 2:3] = szc * sinv

    x = x_ref[...]
    y = y_ref[...]
    z = z_ref[...]
    s = x * x + y * y + z * z
    inv = _rsqrt_refined(s)
    xn = x * inv
    yn = y * inv
    zn = z * inv

    xnb = xn.astype(jnp.bfloat16)
    ynb = yn.astype(jnp.bfloat16)
    znb = zn.astype(jnp.bfloat16)

    fmt = fm_ref[...].T
    n = x.shape[0]
    o = fmt.shape[0]
    s_num = supt_ref.shape[0] // o
    cols = []
    for t in range(s_num):
        sx_t = sup_scr[t * o:(t + 1) * o, 0:1].astype(jnp.bfloat16)
        sy_t = sup_scr[t * o:(t + 1) * o, 1:2].astype(jnp.bfloat16)
        sz_t = sup_scr[t * o:(t + 1) * o, 2:3].astype(jnp.bfloat16)
        m = None
        for j in range(n):
            th = (xnb[j:j + 1] * sx_t + ynb[j:j + 1] * sy_t
                  + znb[j:j + 1] * sz_t)
            m = th if m is None else jnp.maximum(m, th)
        m = jnp.maximum(m.astype(jnp.float32), 0.0)
        cols.append(jnp.sum(fmt * m, axis=0, keepdims=True))
    out_t = jnp.concatenate(cols, axis=0)
    out_ref[...] = out_t.T + bias_ref[...]


def _tc_compute(xt, yt, zt, fm, supt_pad, bias2d, rows_block=512):
    n, bv = xt.shape
    o = fm.shape[1]
    s_num = bias2d.shape[1]
    dir_spec = pl.BlockSpec((n, rows_block), lambda i: (0, i))
    return pl.pallas_call(
        _tc_body,
        grid=(bv // rows_block,),
        in_specs=[
            dir_spec,
            dir_spec,
            dir_spec,
            pl.BlockSpec((rows_block, o), lambda i: (i, 0)),
            pl.BlockSpec((s_num * o, 8), lambda i: (0, 0)),
            pl.BlockSpec((1, s_num), lambda i: (0, 0)),
        ],
        out_specs=pl.BlockSpec((rows_block, s_num), lambda i: (i, 0)),
        out_shape=jax.ShapeDtypeStruct((bv, s_num), jnp.float32),
        scratch_shapes=[pltpu.VMEM((s_num * o, 8), jnp.float32)],
    )(xt, yt, zt, fm, supt_pad, bias2d)


def kernel(neighbor_index, vertices, feature_map, directions, bias):
    bs, v, n = neighbor_index.shape
    o = feature_map.shape[-1]
    s_num = directions.shape[1] // o
    bv = bs * v

    idx = neighbor_index.astype(jnp.int32)
    gidx = idx + (jnp.arange(bs, dtype=jnp.int32) * v)[:, None, None]
    gidx_j = jnp.transpose(gidx, (2, 0, 1)).reshape(-1)

    vflat = vertices.reshape(bv, 3)
    dx, dy, dz = _sc_dirs(vflat[:, 0], vflat[:, 1], vflat[:, 2],
                          gidx_j, n * bv)

    fm = feature_map.reshape(bv, o)
    supt_pad = jnp.zeros((s_num * o, 8), jnp.float32).at[:, :3].set(
        directions.T)
    bias2d = bias.reshape(1, s_num)

    out = _tc_compute(dx.reshape(n, bv), dy.reshape(n, bv),
                      dz.reshape(n, bv), fm, supt_pad, bias2d)
    return out.reshape(bs, v, s_num)


# speedup vs baseline: 1.6917x; 1.0189x over previous
import functools

import jax
import jax.numpy as jnp
from jax import lax
from jax.experimental import pallas as pl
from jax.experimental.pallas import tpu as pltpu

try:
    from jax.experimental.pallas import tpu_sc as plsc
except ImportError:
    plsc = None

_L = 16


def _sc_dirs(xs, ys, zs, gidx_j, num_rows):
    info = plsc.get_sparse_core_info()
    nw = info.num_cores * info.num_subcores
    per_w = num_rows // nw
    steps = per_w // _L
    bv = xs.shape[0]
    mesh = plsc.VectorSubcoreMesh(core_axis_name="c", subcore_axis_name="s")
    plane = jax.ShapeDtypeStruct((num_rows,), jnp.float32)

    @functools.partial(
        pl.kernel,
        mesh=mesh,
        out_type=(plane, plane, plane),
        compiler_params=pltpu.CompilerParams(needs_layout_passes=False),
        scratch_types=[
            pltpu.VMEM((bv,), jnp.float32),
            pltpu.VMEM((bv,), jnp.float32),
            pltpu.VMEM((bv,), jnp.float32),
            pltpu.VMEM((per_w,), jnp.int32),
            pltpu.VMEM((per_w,), jnp.float32),
            pltpu.VMEM((per_w,), jnp.float32),
            pltpu.VMEM((per_w,), jnp.float32),
        ],
    )
    def k(xs_hbm, ys_hbm, zs_hbm, idx_hbm, ox_hbm, oy_hbm, oz_hbm,
          xs_v, ys_v, zs_v, idx_v, dx_v, dy_v, dz_v):
        wid = lax.axis_index("s") * info.num_cores + lax.axis_index("c")
        base = wid * per_w
        gbase = lax.rem(base, bv)
        pltpu.sync_copy(xs_hbm, xs_v)
        pltpu.sync_copy(ys_hbm, ys_v)
        pltpu.sync_copy(zs_hbm, zs_v)
        pltpu.sync_copy(idx_hbm.at[pl.ds(base, per_w)], idx_v)

        @functools.partial(plsc.parallel_loop(0, steps, unroll=4))
        def _(i):
            off = i * _L
            iv = idx_v[pl.ds(off, _L)]
            gx = plsc.load_gather(xs_v, [iv])
            gy = plsc.load_gather(ys_v, [iv])
            gz = plsc.load_gather(zs_v, [iv])
            coff = gbase + off
            dx_v[pl.ds(off, _L)] = gx - xs_v[pl.ds(coff, _L)]
            dy_v[pl.ds(off, _L)] = gy - ys_v[pl.ds(coff, _L)]
            dz_v[pl.ds(off, _L)] = gz - zs_v[pl.ds(coff, _L)]
        pltpu.sync_copy(dx_v, ox_hbm.at[pl.ds(base, per_w)])
        pltpu.sync_copy(dy_v, oy_hbm.at[pl.ds(base, per_w)])
        pltpu.sync_copy(dz_v, oz_hbm.at[pl.ds(base, per_w)])

    return k(xs, ys, zs, gidx_j)


def _rsqrt_refined(s):
    r = lax.rsqrt(jnp.maximum(s, 1e-30))
    r = r * (1.5 - 0.5 * s * r * r)
    return jnp.where(s > 0.0, r, 0.0)


def _tc_body(x_ref, y_ref, z_ref, fm_ref, supt_ref, bias_ref, out_ref,
             sup_scr):
    @pl.when(pl.program_id(0) == 0)
    def _():
        supt = supt_ref[...]
        sxc = supt[:, 0:1]
        syc = supt[:, 1:2]
        szc = supt[:, 2:3]
        s2 = sxc * sxc + syc * syc + szc * szc
        sinv = _rsqrt_refined(s2)
        sup_scr[:, 0:1] = sxc * sinv
        sup_scr[:, 1:2] = syc * sinv
        sup_scr[:, 2:3] = szc * sinv

    x = x_ref[...]
    y = y_ref[...]
    z = z_ref[...]
    s = x * x + y * y + z * z
    inv = _rsqrt_refined(s)
    xn = x * inv
    yn = y * inv
    zn = z * inv

    xnb = xn.astype(jnp.bfloat16)
    ynb = yn.astype(jnp.bfloat16)
    znb = zn.astype(jnp.bfloat16)

    fmt = fm_ref[...].T
    n = x.shape[0]
    o = fmt.shape[0]
    s_num = supt_ref.shape[0] // o
    cols = []
    for t in range(s_num):
        sx_t = sup_scr[t * o:(t + 1) * o, 0:1].astype(jnp.bfloat16)
        sy_t = sup_scr[t * o:(t + 1) * o, 1:2].astype(jnp.bfloat16)
        sz_t = sup_scr[t * o:(t + 1) * o, 2:3].astype(jnp.bfloat16)
        m = None
        for j in range(n):
            th = (xnb[j:j + 1] * sx_t + ynb[j:j + 1] * sy_t
                  + znb[j:j + 1] * sz_t)
            m = th if m is None else jnp.maximum(m, th)
        m = jnp.maximum(m.astype(jnp.float32), 0.0)
        cols.append(jnp.sum(fmt * m, axis=0, keepdims=True))
    out_t = jnp.concatenate(cols, axis=0)
    out_ref[...] = out_t.T + bias_ref[...]


def _tc_compute(xt, yt, zt, fm, supt_pad, bias2d, rows_block=1024):
    n, bv = xt.shape
    o = fm.shape[1]
    s_num = bias2d.shape[1]
    dir_spec = pl.BlockSpec((n, rows_block), lambda i: (0, i))
    return pl.pallas_call(
        _tc_body,
        grid=(bv // rows_block,),
        in_specs=[
            dir_spec,
            dir_spec,
            dir_spec,
            pl.BlockSpec((rows_block, o), lambda i: (i, 0)),
            pl.BlockSpec((s_num * o, 8), lambda i: (0, 0)),
            pl.BlockSpec((1, s_num), lambda i: (0, 0)),
        ],
        out_specs=pl.BlockSpec((rows_block, s_num), lambda i: (i, 0)),
        out_shape=jax.ShapeDtypeStruct((bv, s_num), jnp.float32),
        scratch_shapes=[pltpu.VMEM((s_num * o, 8), jnp.float32)],
    )(xt, yt, zt, fm, supt_pad, bias2d)


def kernel(neighbor_index, vertices, feature_map, directions, bias):
    bs, v, n = neighbor_index.shape
    o = feature_map.shape[-1]
    s_num = directions.shape[1] // o
    bv = bs * v

    idx = neighbor_index.astype(jnp.int32)
    gidx = idx + (jnp.arange(bs, dtype=jnp.int32) * v)[:, None, None]
    gidx_j = jnp.transpose(gidx, (2, 0, 1)).reshape(-1)

    vflat = vertices.reshape(bv, 3)
    dx, dy, dz = _sc_dirs(vflat[:, 0], vflat[:, 1], vflat[:, 2],
                          gidx_j, n * bv)

    fm = feature_map.reshape(bv, o)
    supt_pad = jnp.zeros((s_num * o, 8), jnp.float32).at[:, :3].set(
        directions.T)
    bias2d = bias.reshape(1, s_num)

    out = _tc_compute(dx.reshape(n, bv), dy.reshape(n, bv),
                      dz.reshape(n, bv), fm, supt_pad, bias2d)
    return out.reshape(bs, v, s_num)


# speedup vs baseline: 1.7427x; 1.0302x over previous
import functools

import jax
import jax.numpy as jnp
from jax import lax
from jax.experimental import pallas as pl
from jax.experimental.pallas import tpu as pltpu

try:
    from jax.experimental.pallas import tpu_sc as plsc
except ImportError:
    plsc = None

_L = 16


def _sc_dirs(xs, ys, zs, gidx_j, num_rows):
    info = plsc.get_sparse_core_info()
    nw = info.num_cores * info.num_subcores
    per_w = num_rows // nw
    steps = per_w // _L
    bv = xs.shape[0]
    mesh = plsc.VectorSubcoreMesh(core_axis_name="c", subcore_axis_name="s")
    plane = jax.ShapeDtypeStruct((num_rows,), jnp.float32)

    @functools.partial(
        pl.kernel,
        mesh=mesh,
        out_type=(plane, plane, plane),
        compiler_params=pltpu.CompilerParams(needs_layout_passes=False),
        scratch_types=[
            pltpu.VMEM((bv,), jnp.float32),
            pltpu.VMEM((bv,), jnp.float32),
            pltpu.VMEM((bv,), jnp.float32),
            pltpu.VMEM((per_w,), jnp.int32),
            pltpu.VMEM((per_w,), jnp.float32),
            pltpu.VMEM((per_w,), jnp.float32),
            pltpu.VMEM((per_w,), jnp.float32),
            pltpu.SemaphoreType.DMA,
        ],
    )
    def k(xs_hbm, ys_hbm, zs_hbm, idx_hbm, ox_hbm, oy_hbm, oz_hbm,
          xs_v, ys_v, zs_v, idx_v, dx_v, dy_v, dz_v, sem):
        wid = lax.axis_index("s") * info.num_cores + lax.axis_index("c")
        base = wid * per_w
        gbase = lax.rem(base, bv)
        ins = [
            pltpu.async_copy(xs_hbm, xs_v, sem),
            pltpu.async_copy(ys_hbm, ys_v, sem),
            pltpu.async_copy(zs_hbm, zs_v, sem),
            pltpu.async_copy(idx_hbm.at[pl.ds(base, per_w)], idx_v, sem),
        ]
        for cp in ins:
            cp.wait()

        @functools.partial(plsc.parallel_loop(0, steps, unroll=4))
        def _(i):
            off = i * _L
            iv = idx_v[pl.ds(off, _L)]
            gx = plsc.load_gather(xs_v, [iv])
            gy = plsc.load_gather(ys_v, [iv])
            gz = plsc.load_gather(zs_v, [iv])
            coff = gbase + off
            dx_v[pl.ds(off, _L)] = gx - xs_v[pl.ds(coff, _L)]
            dy_v[pl.ds(off, _L)] = gy - ys_v[pl.ds(coff, _L)]
            dz_v[pl.ds(off, _L)] = gz - zs_v[pl.ds(coff, _L)]
        outs = [
            pltpu.async_copy(dx_v, ox_hbm.at[pl.ds(base, per_w)], sem),
            pltpu.async_copy(dy_v, oy_hbm.at[pl.ds(base, per_w)], sem),
            pltpu.async_copy(dz_v, oz_hbm.at[pl.ds(base, per_w)], sem),
        ]
        for cp in outs:
            cp.wait()

    return k(xs, ys, zs, gidx_j)


def _rsqrt_refined(s):
    r = lax.rsqrt(jnp.maximum(s, 1e-30))
    r = r * (1.5 - 0.5 * s * r * r)
    return jnp.where(s > 0.0, r, 0.0)


def _tc_body(x_ref, y_ref, z_ref, fm_ref, supt_ref, bias_ref, out_ref,
             sup_scr):
    @pl.when(pl.program_id(0) == 0)
    def _():
        supt = supt_ref[...]
        sxc = supt[:, 0:1]
        syc = supt[:, 1:2]
        szc = supt[:, 2:3]
        s2 = sxc * sxc + syc * syc + szc * szc
        sinv = _rsqrt_refined(s2)
        sup_scr[:, 0:1] = sxc * sinv
        sup_scr[:, 1:2] = syc * sinv
        sup_scr[:, 2:3] = szc * sinv

    x = x_ref[...]
    y = y_ref[...]
    z = z_ref[...]
    s = x * x + y * y + z * z
    inv = _rsqrt_refined(s)
    xn = x * inv
    yn = y * inv
    zn = z * inv

    xnb = xn.astype(jnp.bfloat16)
    ynb = yn.astype(jnp.bfloat16)
    znb = zn.astype(jnp.bfloat16)

    fmt = fm_ref[...].T
    n = x.shape[0]
    o = fmt.shape[0]
    s_num = supt_ref.shape[0] // o
    cols = []
    for t in range(s_num):
        sx_t = sup_scr[t * o:(t + 1) * o, 0:1].astype(jnp.bfloat16)
        sy_t = sup_scr[t * o:(t + 1) * o, 1:2].astype(jnp.bfloat16)
        sz_t = sup_scr[t * o:(t + 1) * o, 2:3].astype(jnp.bfloat16)
        m = None
        for j in range(n):
            th = (xnb[j:j + 1] * sx_t + ynb[j:j + 1] * sy_t
                  + znb[j:j + 1] * sz_t)
            m = th if m is None else jnp.maximum(m, th)
        m = jnp.maximum(m.astype(jnp.float32), 0.0)
        cols.append(jnp.sum(fmt * m, axis=0, keepdims=True))
    out_t = jnp.concatenate(cols, axis=0)
    out_ref[...] = out_t.T + bias_ref[...]


def _tc_compute(xt, yt, zt, fm, supt_pad, bias2d, rows_block=1024):
    n, bv = xt.shape
    o = fm.shape[1]
    s_num = bias2d.shape[1]
    dir_spec = pl.BlockSpec((n, rows_block), lambda i: (0, i))
    return pl.pallas_call(
        _tc_body,
        grid=(bv // rows_block,),
        in_specs=[
            dir_spec,
            dir_spec,
            dir_spec,
            pl.BlockSpec((rows_block, o), lambda i: (i, 0)),
            pl.BlockSpec((s_num * o, 8), lambda i: (0, 0)),
            pl.BlockSpec((1, s_num), lambda i: (0, 0)),
        ],
        out_specs=pl.BlockSpec((rows_block, s_num), lambda i: (i, 0)),
        out_shape=jax.ShapeDtypeStruct((bv, s_num), jnp.float32),
        scratch_shapes=[pltpu.VMEM((s_num * o, 8), jnp.float32)],
    )(xt, yt, zt, fm, supt_pad, bias2d)


def kernel(neighbor_index, vertices, feature_map, directions, bias):
    bs, v, n = neighbor_index.shape
    o = feature_map.shape[-1]
    s_num = directions.shape[1] // o
    bv = bs * v

    idx = neighbor_index.astype(jnp.int32)
    gidx = idx + (jnp.arange(bs, dtype=jnp.int32) * v)[:, None, None]
    gidx_j = jnp.transpose(gidx, (2, 0, 1)).reshape(-1)

    vflat = vertices.reshape(bv, 3)
    dx, dy, dz = _sc_dirs(vflat[:, 0], vflat[:, 1], vflat[:, 2],
                          gidx_j, n * bv)

    fm = feature_map.reshape(bv, o)
    supt_pad = jnp.zeros((s_num * o, 8), jnp.float32).at[:, :3].set(
        directions.T)
    bias2d = bias.reshape(1, s_num)

    out = _tc_compute(dx.reshape(n, bv), dy.reshape(n, bv),
                      dz.reshape(n, bv), fm, supt_pad, bias2d)
    return out.reshape(bs, v, s_num)
